# baseline (speedup 1.0000x reference)
import functools

import jax
import jax.numpy as jnp
from jax import lax
from jax.experimental import pallas as pl
from jax.experimental.pallas import tpu as pltpu
from jax.experimental.pallas import tpu_sc as plsc

N = 10000
E = 320000
H = 64
C = 16
EPS = 1.0

ROW_BLK = 1024
CH = 4000
NW = 32
NSC = 16
PACK_SHIFT = 14


def _sc_agg_body(fdim, split_cores, packedE, aggT, xcol, aggcol, pbufs, sems):
    c = lax.axis_index("c")
    s = lax.axis_index("s")
    if split_cores:
        fpt = fdim // NSC
        f0 = s * fpt
        ebase = c * (E // 2)
        nchunk = (E // 2) // CH
        out_off = c * (fdim * N)
    else:
        fpt = fdim // NW
        f0 = (s * 2 + c) * fpt
        ebase = 0
        nchunk = E // CH
        out_off = 0

    pltpu.async_copy(packedE.at[pl.ds(ebase, CH)], pbufs[0], sems[0])

    def zero_body(i, _):
        aggcol[pl.ds(i * 16, 16)] = jnp.zeros((16,), jnp.float32)
        return 0

    lax.fori_loop(0, fpt * N // 16, zero_body, 0)

    def chunk_pair(ci2, _):
        for b in range(2):
            ci = ci2 * 2 + b
            pltpu.make_async_copy(
                packedE.at[pl.ds(ebase + ci * CH, CH)], pbufs[b], sems[b]
            ).wait()

            @pl.when(ci + 1 < nchunk)
            def _():
                pltpu.async_copy(
                    packedE.at[pl.ds(ebase + (ci + 1) * CH, CH)],
                    pbufs[1 - b], sems[1 - b])

            def grp(g, _):
                for u in range(2):
                    p16 = pbufs[b][pl.ds(g * 32 + u * 16, 16)]
                    s16 = lax.bitwise_and(p16, (1 << PACK_SHIFT) - 1)
                    d16 = lax.shift_right_logical(p16, PACK_SHIFT)
                    for f in range(fpt):
                        v = plsc.load_gather(xcol, [s16 + (f * N)])
                        plsc.addupdate_scatter(aggcol, [d16 + (f * N)], v)
                return 0

            lax.fori_loop(0, CH // 32, grp, 0)
        return 0

    lax.fori_loop(0, nchunk // 2, chunk_pair, 0)
    pltpu.sync_copy(aggcol, aggT.at[pl.ds(out_off + f0 * N, fpt * N)])


def _sc_xload_body(fdim, split_cores, xT, xcol):
    c = lax.axis_index("c")
    s = lax.axis_index("s")
    if split_cores:
        f0 = s * (fdim // NSC)
        fpt = fdim // NSC
    else:
        f0 = (s * 2 + c) * (fdim // NW)
        fpt = fdim // NW
    pltpu.sync_copy(xT.at[pl.ds(f0 * N, fpt * N)], xcol)


def _sc_aggregate(xT, packed, split_cores):
    fdim = xT.shape[0]
    fpt = fdim // (NSC if split_cores else NW)
    ncopies = 2 if split_cores else 1
    mesh = plsc.VectorSubcoreMesh(core_axis_name="c", subcore_axis_name="s",
                                  num_cores=2, num_subcores=16)

    def body(xT_hbm, packedE, aggT, xcol, aggcol, pbuf0, pbuf1, sem0, sem1):
        _sc_xload_body(fdim, split_cores, xT_hbm, xcol)
        _sc_agg_body(fdim, split_cores, packedE, aggT, xcol, aggcol,
                     (pbuf0, pbuf1), (sem0, sem1))

    run = pl.kernel(
        body,
        out_type=jax.ShapeDtypeStruct((ncopies * fdim * N,), jnp.float32),
        mesh=mesh,
        scratch_types=[
            pltpu.VMEM((fpt * N,), jnp.float32),
            pltpu.VMEM((fpt * N,), jnp.float32),
            pltpu.VMEM((CH,), jnp.int32),
            pltpu.VMEM((CH,), jnp.int32),
            pltpu.SemaphoreType.DMA,
            pltpu.SemaphoreType.DMA,
        ],
        compiler_params=pltpu.CompilerParams(needs_layout_passes=False),
    )
    out = run(xT.reshape(fdim * N), packed)
    if split_cores:
        return out.reshape(2, fdim, N)
    return out.reshape(1, fdim, N)


def _pack_body(s_ref, d_ref, p_ref):
    p_ref[...] = jnp.bitwise_or(s_ref[...],
                                jnp.left_shift(d_ref[...], PACK_SHIFT))


def _pack_edges(src, dst):
    s2 = src.reshape(E // 128, 128)
    d2 = dst.reshape(E // 128, 128)
    p = pl.pallas_call(
        _pack_body,
        in_specs=[
            pl.BlockSpec((E // 128, 128), lambda: (0, 0)),
            pl.BlockSpec((E // 128, 128), lambda: (0, 0)),
        ],
        out_specs=pl.BlockSpec((E // 128, 128), lambda: (0, 0)),
        out_shape=jax.ShapeDtypeStruct((E // 128, 128), jnp.int32),
    )(s2, d2)
    return p.reshape(E)


def _transpose_body(x_ref, xT_ref):
    xT_ref[...] = x_ref[...].T


def _transpose(x):
    n, f = x.shape
    grid = (n + ROW_BLK - 1) // ROW_BLK
    return pl.pallas_call(
        _transpose_body,
        grid=(grid,),
        in_specs=[pl.BlockSpec((ROW_BLK, f), lambda i: (i, 0))],
        out_specs=pl.BlockSpec((f, ROW_BLK), lambda i: (0, i)),
        out_shape=jax.ShapeDtypeStruct((f, n), jnp.float32),
    )(x)


def _layer_body(nagg, x_ref, agg_ref, w1_ref, b1_ref, w2_ref, b2_ref,
                h_ref, hT_ref):
    agg = agg_ref[...]
    aggT = agg[0]
    for a in range(1, nagg):
        aggT = aggT + agg[a]
    u = (1.0 + EPS) * x_ref[...] + aggT.T
    t1 = jnp.dot(u, w1_ref[...], preferred_element_type=jnp.float32) + b1_ref[...]
    z = jnp.maximum(t1, 0.0)
    t2 = jnp.dot(z, w2_ref[...], preferred_element_type=jnp.float32) + b2_ref[...]
    s = jax.nn.sigmoid(t2)
    h = (s > 0.5).astype(jnp.float32)
    h_ref[...] = h
    hT_ref[...] = h.T


def _layer_mlp(x, aggTs, w1, b1, w2, b2):
    f = x.shape[1]
    nagg = aggTs.shape[0]
    grid = (N + ROW_BLK - 1) // ROW_BLK
    return pl.pallas_call(
        functools.partial(_layer_body, nagg),
        grid=(grid,),
        in_specs=[
            pl.BlockSpec((ROW_BLK, f), lambda i: (i, 0)),
            pl.BlockSpec((nagg, f, ROW_BLK), lambda i: (0, 0, i)),
            pl.BlockSpec((f, H), lambda i: (0, 0)),
            pl.BlockSpec((H,), lambda i: (0,)),
            pl.BlockSpec((H, H), lambda i: (0, 0)),
            pl.BlockSpec((H,), lambda i: (0,)),
        ],
        out_specs=[
            pl.BlockSpec((ROW_BLK, H), lambda i: (i, 0)),
            pl.BlockSpec((H, ROW_BLK), lambda i: (0, i)),
        ],
        out_shape=[
            jax.ShapeDtypeStruct((N, H), jnp.float32),
            jax.ShapeDtypeStruct((H, N), jnp.float32),
        ],
    )(x, aggTs, w1, b1, w2, b2)


def _head_body(h0_ref, h1_ref, h2_ref, fc1w_ref, fc1b_ref, fc2w_ref, fc2b_ref,
               out_ref, sum_ref, max_ref):
    i = pl.program_id(0)
    nblk = pl.num_programs(0)
    xh = jnp.concatenate([h0_ref[...], h1_ref[...], h2_ref[...]], axis=1)
    row = i * ROW_BLK + jax.lax.broadcasted_iota(jnp.int32, xh.shape, 0)
    valid = row < N
    xs = jnp.where(valid, xh, 0.0)
    xm = jnp.where(valid, xh, -jnp.inf)
    psum = jnp.sum(xs, axis=0, keepdims=True)
    pmax = jnp.max(xm, axis=0, keepdims=True)

    @pl.when(i == 0)
    def _():
        sum_ref[...] = psum
        max_ref[...] = pmax

    @pl.when(i > 0)
    def _():
        sum_ref[...] += psum
        max_ref[...] = jnp.maximum(max_ref[...], pmax)

    @pl.when(i == nblk - 1)
    def _():
        tsum = sum_ref[...]
        feat = jnp.concatenate([tsum / N, max_ref[...], tsum], axis=1)
        o1 = jnp.maximum(
            jnp.dot(feat, fc1w_ref[...], preferred_element_type=jnp.float32)
            + fc1b_ref[...], 0.0)
        o2 = (jnp.dot(o1, fc2w_ref[...], preferred_element_type=jnp.float32)
              + fc2b_ref[...])
        out_ref[...] = jax.nn.sigmoid(o2)


def _head(h0, h1, h2, fc1_W, fc1_b, fc2_W, fc2_b):
    grid = (N + ROW_BLK - 1) // ROW_BLK
    out, _, _ = pl.pallas_call(
        _head_body,
        grid=(grid,),
        in_specs=[
            pl.BlockSpec((ROW_BLK, H), lambda i: (i, 0)),
            pl.BlockSpec((ROW_BLK, H), lambda i: (i, 0)),
            pl.BlockSpec((ROW_BLK, H), lambda i: (i, 0)),
            pl.BlockSpec((3 * 3 * H, H), lambda i: (0, 0)),
            pl.BlockSpec((H,), lambda i: (0,)),
            pl.BlockSpec((H, C), lambda i: (0, 0)),
            pl.BlockSpec((C,), lambda i: (0,)),
        ],
        out_specs=[
            pl.BlockSpec((1, C), lambda i: (0, 0)),
            pl.BlockSpec((1, 3 * H), lambda i: (0, 0)),
            pl.BlockSpec((1, 3 * H), lambda i: (0, 0)),
        ],
        out_shape=[
            jax.ShapeDtypeStruct((1, C), jnp.float32),
            jax.ShapeDtypeStruct((1, 3 * H), jnp.float32),
            jax.ShapeDtypeStruct((1, 3 * H), jnp.float32),
        ],
    )(h0, h1, h2, fc1_W, fc1_b, fc2_W, fc2_b)
    return out


def kernel(x, edge_index, W1_0, b1_0, W2_0, b2_0, W1_1, b1_1, W2_1, b2_1,
           W1_2, b1_2, W2_2, b2_2, fc1_W, fc1_b, fc2_W, fc2_b):
    src = edge_index[0]
    dst = edge_index[1]
    packed = _pack_edges(src, dst)
    xT = _transpose(x)
    aggT0 = _sc_aggregate(xT, packed, split_cores=False)
    h0, h0T = _layer_mlp(x, aggT0, W1_0, b1_0, W2_0, b2_0)
    aggT1 = _sc_aggregate(h0T, packed, split_cores=True)
    h1, h1T = _layer_mlp(h0, aggT1, W1_1, b1_1, W2_1, b2_1)
    aggT2 = _sc_aggregate(h1T, packed, split_cores=True)
    h2, _ = _layer_mlp(h1, aggT2, W1_2, b1_2, W2_2, b2_2)
    return _head(h0, h1, h2, fc1_W, fc1_b, fc2_W, fc2_b)


# speedup vs baseline: 1.1020x; 1.1020x over previous
import functools

import jax
import jax.numpy as jnp
from jax import lax
from jax.experimental import pallas as pl
from jax.experimental.pallas import tpu as pltpu
from jax.experimental.pallas import tpu_sc as plsc

N = 10000
E = 320000
H = 64
C = 16
EPS = 1.0

ROW_BLK = 1024
CH = 4000
NW = 32
NSC = 16
PACK_SHIFT = 14


def _sc_agg_body(fdim, split_cores, packedE, aggT, xcol, aggcol, pbufs, sems):
    c = lax.axis_index("c")
    s = lax.axis_index("s")
    if split_cores:
        fpt = fdim // NSC
        f0 = s * fpt
        ebase = c * (E // 2)
        nchunk = (E // 2) // CH
        out_off = c * (fdim * N)
    else:
        fpt = fdim // NW
        f0 = (s * 2 + c) * fpt
        ebase = 0
        nchunk = E // CH
        out_off = 0

    pltpu.async_copy(packedE.at[pl.ds(ebase, CH)], pbufs[0], sems[0])

    def zero_body(i, _):
        aggcol[pl.ds(i * 16, 16)] = jnp.zeros((16,), jnp.float32)
        return 0

    lax.fori_loop(0, fpt * N // 16, zero_body, 0)

    def chunk_pair(ci2, _):
        for b in range(2):
            ci = ci2 * 2 + b
            pltpu.make_async_copy(
                packedE.at[pl.ds(ebase + ci * CH, CH)], pbufs[b], sems[b]
            ).wait()

            @pl.when(ci + 1 < nchunk)
            def _():
                pltpu.async_copy(
                    packedE.at[pl.ds(ebase + (ci + 1) * CH, CH)],
                    pbufs[1 - b], sems[1 - b])

            def grp(g, _):
                for u in range(2):
                    p16 = pbufs[b][pl.ds(g * 32 + u * 16, 16)]
                    s16 = lax.bitwise_and(p16, (1 << PACK_SHIFT) - 1)
                    d16 = lax.shift_right_logical(p16, PACK_SHIFT)
                    for f in range(fpt):
                        v = plsc.load_gather(xcol, [s16 + (f * N)])
                        plsc.addupdate_scatter(aggcol, [d16 + (f * N)], v)
                return 0

            lax.fori_loop(0, CH // 32, grp, 0)
        return 0

    lax.fori_loop(0, nchunk // 2, chunk_pair, 0)
    pltpu.sync_copy(aggcol, aggT.at[pl.ds(out_off + f0 * N, fpt * N)])


def _sc_agg_bits_body(packedE, hbits0, hbits1, aggT, wtab, aggcol,
                      pbuf0, pbuf1, sem0, sem1):
    fdim = H
    fpt = fdim // NSC
    pbufs = (pbuf0, pbuf1)
    sems = (sem0, sem1)
    c = lax.axis_index("c")
    s = lax.axis_index("s")
    f0 = s * fpt
    ebase = c * (E // 2)
    nchunk = (E // 2) // CH
    out_off = c * (fdim * N)

    pltpu.async_copy(packedE.at[pl.ds(ebase, CH)], pbufs[0], sems[0])

    @pl.when(f0 < 32)
    def _():
        pltpu.sync_copy(hbits0, wtab)

    @pl.when(f0 >= 32)
    def _():
        pltpu.sync_copy(hbits1, wtab)

    fb0 = lax.rem(f0, 32)

    def zero_body(i, _):
        aggcol[pl.ds(i * 16, 16)] = jnp.zeros((16,), jnp.float32)
        return 0

    lax.fori_loop(0, fpt * N // 16, zero_body, 0)

    ones = jnp.ones((16,), jnp.float32)

    def chunk_pair(ci2, _):
        for b in range(2):
            ci = ci2 * 2 + b
            pltpu.make_async_copy(
                packedE.at[pl.ds(ebase + ci * CH, CH)], pbufs[b], sems[b]
            ).wait()

            @pl.when(ci + 1 < nchunk)
            def _():
                pltpu.async_copy(
                    packedE.at[pl.ds(ebase + (ci + 1) * CH, CH)],
                    pbufs[1 - b], sems[1 - b])

            def grp(g, _):
                p16 = pbufs[b][pl.ds(g * 16, 16)]
                s16 = lax.bitwise_and(p16, (1 << PACK_SHIFT) - 1)
                d16 = lax.shift_right_logical(p16, PACK_SHIFT)
                w16 = plsc.load_gather(wtab, [s16])
                wsh = lax.shift_right_logical(
                    w16, lax.broadcast_in_dim(fb0, (16,), ()))
                for f in range(fpt):
                    bit = lax.bitwise_and(
                        lax.shift_right_logical(wsh, f), 1)
                    plsc.addupdate_scatter(aggcol, [d16 + (f * N)], ones,
                                           mask=bit == 1)
                return 0

            lax.fori_loop(0, CH // 16, grp, 0)
        return 0

    lax.fori_loop(0, nchunk // 2, chunk_pair, 0)
    pltpu.sync_copy(aggcol, aggT.at[pl.ds(out_off + f0 * N, fpt * N)])


def _sc_aggregate_bits(hbits0, hbits1, packed):
    fpt = H // NSC
    mesh = plsc.VectorSubcoreMesh(core_axis_name="c", subcore_axis_name="s",
                                  num_cores=2, num_subcores=16)
    run = pl.kernel(
        _sc_agg_bits_body,
        out_type=jax.ShapeDtypeStruct((2 * H * N,), jnp.float32),
        mesh=mesh,
        scratch_types=[
            pltpu.VMEM((N,), jnp.int32),
            pltpu.VMEM((fpt * N,), jnp.float32),
            pltpu.VMEM((CH,), jnp.int32),
            pltpu.VMEM((CH,), jnp.int32),
            pltpu.SemaphoreType.DMA,
            pltpu.SemaphoreType.DMA,
        ],
        compiler_params=pltpu.CompilerParams(needs_layout_passes=False),
    )
    return run(packed, hbits0, hbits1).reshape(2, H, N)


def _sc_xload_body(fdim, split_cores, xT, xcol):
    c = lax.axis_index("c")
    s = lax.axis_index("s")
    if split_cores:
        f0 = s * (fdim // NSC)
        fpt = fdim // NSC
    else:
        f0 = (s * 2 + c) * (fdim // NW)
        fpt = fdim // NW
    pltpu.sync_copy(xT.at[pl.ds(f0 * N, fpt * N)], xcol)


def _sc_aggregate(xT, packed, split_cores):
    fdim = xT.shape[0]
    fpt = fdim // (NSC if split_cores else NW)
    ncopies = 2 if split_cores else 1
    mesh = plsc.VectorSubcoreMesh(core_axis_name="c", subcore_axis_name="s",
                                  num_cores=2, num_subcores=16)

    def body(xT_hbm, packedE, aggT, xcol, aggcol, pbuf0, pbuf1, sem0, sem1):
        _sc_xload_body(fdim, split_cores, xT_hbm, xcol)
        _sc_agg_body(fdim, split_cores, packedE, aggT, xcol, aggcol,
                     (pbuf0, pbuf1), (sem0, sem1))

    run = pl.kernel(
        body,
        out_type=jax.ShapeDtypeStruct((ncopies * fdim * N,), jnp.float32),
        mesh=mesh,
        scratch_types=[
            pltpu.VMEM((fpt * N,), jnp.float32),
            pltpu.VMEM((fpt * N,), jnp.float32),
            pltpu.VMEM((CH,), jnp.int32),
            pltpu.VMEM((CH,), jnp.int32),
            pltpu.SemaphoreType.DMA,
            pltpu.SemaphoreType.DMA,
        ],
        compiler_params=pltpu.CompilerParams(needs_layout_passes=False),
    )
    out = run(xT.reshape(fdim * N), packed)
    if split_cores:
        return out.reshape(2, fdim, N)
    return out.reshape(1, fdim, N)


def _pack_body(s_ref, d_ref, p_ref):
    p_ref[...] = jnp.bitwise_or(s_ref[...],
                                jnp.left_shift(d_ref[...], PACK_SHIFT))


def _pack_edges(src, dst):
    s2 = src.reshape(E // 128, 128)
    d2 = dst.reshape(E // 128, 128)
    p = pl.pallas_call(
        _pack_body,
        in_specs=[
            pl.BlockSpec((E // 128, 128), lambda: (0, 0)),
            pl.BlockSpec((E // 128, 128), lambda: (0, 0)),
        ],
        out_specs=pl.BlockSpec((E // 128, 128), lambda: (0, 0)),
        out_shape=jax.ShapeDtypeStruct((E // 128, 128), jnp.int32),
    )(s2, d2)
    return p.reshape(E)


def _transpose_body(x_ref, xT_ref):
    xT_ref[...] = x_ref[...].T


def _transpose(x):
    n, f = x.shape
    grid = (n + ROW_BLK - 1) // ROW_BLK
    return pl.pallas_call(
        _transpose_body,
        grid=(grid,),
        in_specs=[pl.BlockSpec((ROW_BLK, f), lambda i: (i, 0))],
        out_specs=pl.BlockSpec((f, ROW_BLK), lambda i: (0, i)),
        out_shape=jax.ShapeDtypeStruct((f, n), jnp.float32),
    )(x)


def _layer_body(nagg, x_ref, agg_ref, w1_ref, b1_ref, w2_ref, b2_ref,
                h_ref, hb0_ref, hb1_ref):
    agg = agg_ref[...]
    aggT = agg[0]
    for a in range(1, nagg):
        aggT = aggT + agg[a]
    u = (1.0 + EPS) * x_ref[...] + aggT.T
    t1 = jnp.dot(u, w1_ref[...], preferred_element_type=jnp.float32) + b1_ref[...]
    z = jnp.maximum(t1, 0.0)
    t2 = jnp.dot(z, w2_ref[...], preferred_element_type=jnp.float32) + b2_ref[...]
    s = jax.nn.sigmoid(t2)
    h = (s > 0.5).astype(jnp.float32)
    h_ref[...] = h
    hi = (s > 0.5).astype(jnp.int32)
    sh = lax.rem(jax.lax.broadcasted_iota(jnp.int32, hi.shape, 1), 32)
    bits = lax.shift_left(hi, sh)
    hb0_ref[...] = jnp.sum(bits[:, :32], axis=1)
    hb1_ref[...] = jnp.sum(bits[:, 32:], axis=1)


def _layer_mlp(x, aggTs, w1, b1, w2, b2):
    f = x.shape[1]
    nagg = aggTs.shape[0]
    grid = (N + ROW_BLK - 1) // ROW_BLK
    return pl.pallas_call(
        functools.partial(_layer_body, nagg),
        grid=(grid,),
        in_specs=[
            pl.BlockSpec((ROW_BLK, f), lambda i: (i, 0)),
            pl.BlockSpec((nagg, f, ROW_BLK), lambda i: (0, 0, i)),
            pl.BlockSpec((f, H), lambda i: (0, 0)),
            pl.BlockSpec((H,), lambda i: (0,)),
            pl.BlockSpec((H, H), lambda i: (0, 0)),
            pl.BlockSpec((H,), lambda i: (0,)),
        ],
        out_specs=[
            pl.BlockSpec((ROW_BLK, H), lambda i: (i, 0)),
            pl.BlockSpec((ROW_BLK,), lambda i: (i,)),
            pl.BlockSpec((ROW_BLK,), lambda i: (i,)),
        ],
        out_shape=[
            jax.ShapeDtypeStruct((N, H), jnp.float32),
            jax.ShapeDtypeStruct((N,), jnp.int32),
            jax.ShapeDtypeStruct((N,), jnp.int32),
        ],
    )(x, aggTs, w1, b1, w2, b2)


def _head_body(h0_ref, h1_ref, h2_ref, fc1w_ref, fc1b_ref, fc2w_ref, fc2b_ref,
               out_ref, sum_ref, max_ref):
    i = pl.program_id(0)
    nblk = pl.num_programs(0)
    xh = jnp.concatenate([h0_ref[...], h1_ref[...], h2_ref[...]], axis=1)
    row = i * ROW_BLK + jax.lax.broadcasted_iota(jnp.int32, xh.shape, 0)
    valid = row < N
    xs = jnp.where(valid, xh, 0.0)
    xm = jnp.where(valid, xh, -jnp.inf)
    psum = jnp.sum(xs, axis=0, keepdims=True)
    pmax = jnp.max(xm, axis=0, keepdims=True)

    @pl.when(i == 0)
    def _():
        sum_ref[...] = psum
        max_ref[...] = pmax

    @pl.when(i > 0)
    def _():
        sum_ref[...] += psum
        max_ref[...] = jnp.maximum(max_ref[...], pmax)

    @pl.when(i == nblk - 1)
    def _():
        tsum = sum_ref[...]
        feat = jnp.concatenate([tsum / N, max_ref[...], tsum], axis=1)
        o1 = jnp.maximum(
            jnp.dot(feat, fc1w_ref[...], preferred_element_type=jnp.float32)
            + fc1b_ref[...], 0.0)
        o2 = (jnp.dot(o1, fc2w_ref[...], preferred_element_type=jnp.float32)
              + fc2b_ref[...])
        out_ref[...] = jax.nn.sigmoid(o2)


def _head(h0, h1, h2, fc1_W, fc1_b, fc2_W, fc2_b):
    grid = (N + ROW_BLK - 1) // ROW_BLK
    out, _, _ = pl.pallas_call(
        _head_body,
        grid=(grid,),
        in_specs=[
            pl.BlockSpec((ROW_BLK, H), lambda i: (i, 0)),
            pl.BlockSpec((ROW_BLK, H), lambda i: (i, 0)),
            pl.BlockSpec((ROW_BLK, H), lambda i: (i, 0)),
            pl.BlockSpec((3 * 3 * H, H), lambda i: (0, 0)),
            pl.BlockSpec((H,), lambda i: (0,)),
            pl.BlockSpec((H, C), lambda i: (0, 0)),
            pl.BlockSpec((C,), lambda i: (0,)),
        ],
        out_specs=[
            pl.BlockSpec((1, C), lambda i: (0, 0)),
            pl.BlockSpec((1, 3 * H), lambda i: (0, 0)),
            pl.BlockSpec((1, 3 * H), lambda i: (0, 0)),
        ],
        out_shape=[
            jax.ShapeDtypeStruct((1, C), jnp.float32),
            jax.ShapeDtypeStruct((1, 3 * H), jnp.float32),
            jax.ShapeDtypeStruct((1, 3 * H), jnp.float32),
        ],
    )(h0, h1, h2, fc1_W, fc1_b, fc2_W, fc2_b)
    return out


def kernel(x, edge_index, W1_0, b1_0, W2_0, b2_0, W1_1, b1_1, W2_1, b2_1,
           W1_2, b1_2, W2_2, b2_2, fc1_W, fc1_b, fc2_W, fc2_b):
    src = edge_index[0]
    dst = edge_index[1]
    packed = _pack_edges(src, dst)
    xT = _transpose(x)
    aggT0 = _sc_aggregate(xT, packed, split_cores=False)
    h0, h0b0, h0b1 = _layer_mlp(x, aggT0, W1_0, b1_0, W2_0, b2_0)
    aggT1 = _sc_aggregate_bits(h0b0, h0b1, packed)
    h1, h1b0, h1b1 = _layer_mlp(h0, aggT1, W1_1, b1_1, W2_1, b2_1)
    aggT2 = _sc_aggregate_bits(h1b0, h1b1, packed)
    h2, _, _ = _layer_mlp(h1, aggT2, W1_2, b1_2, W2_2, b2_2)
    return _head(h0, h1, h2, fc1_W, fc1_b, fc2_W, fc2_b)


# speedup vs baseline: 1.8464x; 1.6755x over previous
import functools

import jax
import jax.numpy as jnp
from jax import lax
from jax.experimental import pallas as pl
from jax.experimental.pallas import tpu as pltpu
from jax.experimental.pallas import tpu_sc as plsc

N = 10000
E = 320000
H = 64
C = 16
EPS = 1.0

ROW_BLK = 1024
CH = 4000
NW = 32
NSC = 16
PACK_SHIFT = 14


def _sc_agg_body(fdim, split_cores, packedE, aggT, xcol, aggcol, pbufs, sems):
    c = lax.axis_index("c")
    s = lax.axis_index("s")
    if split_cores:
        fpt = fdim // NSC
        f0 = s * fpt
        ebase = c * (E // 2)
        nchunk = (E // 2) // CH
        out_off = c * (fdim * N)
    else:
        fpt = fdim // NW
        f0 = (s * 2 + c) * fpt
        ebase = 0
        nchunk = E // CH
        out_off = 0

    pltpu.async_copy(packedE.at[pl.ds(ebase, CH)], pbufs[0], sems[0])

    def zero_body(i, _):
        aggcol[pl.ds(i * 16, 16)] = jnp.zeros((16,), jnp.float32)
        return 0

    lax.fori_loop(0, fpt * N // 16, zero_body, 0)

    def chunk_pair(ci2, _):
        for b in range(2):
            ci = ci2 * 2 + b
            pltpu.make_async_copy(
                packedE.at[pl.ds(ebase + ci * CH, CH)], pbufs[b], sems[b]
            ).wait()

            @pl.when(ci + 1 < nchunk)
            def _():
                pltpu.async_copy(
                    packedE.at[pl.ds(ebase + (ci + 1) * CH, CH)],
                    pbufs[1 - b], sems[1 - b])

            def grp(g, _):
                vals = []
                dsts = []
                for u in range(2):
                    p16 = pbufs[b][pl.ds(g * 32 + u * 16, 16)]
                    s16 = lax.bitwise_and(p16, (1 << PACK_SHIFT) - 1)
                    d16 = lax.shift_right_logical(p16, PACK_SHIFT)
                    dsts.append(d16)
                    for f in range(fpt):
                        vals.append(plsc.load_gather(xcol, [s16 + (f * N)]))
                for u in range(2):
                    for f in range(fpt):
                        plsc.addupdate_scatter(
                            aggcol, [dsts[u] + (f * N)], vals[u * fpt + f])
                return 0

            lax.fori_loop(0, CH // 32, grp, 0)
        return 0

    lax.fori_loop(0, nchunk // 2, chunk_pair, 0)
    pltpu.sync_copy(aggcol, aggT.at[pl.ds(out_off + f0 * N, fpt * N)])


def _sc_agg_bits_body(packedE, hbits0, hbits1, aggT, wtab, aggcol,
                      pbuf0, pbuf1, sem0, sem1):
    fdim = H
    fpt = fdim // NSC
    pbufs = (pbuf0, pbuf1)
    sems = (sem0, sem1)
    c = lax.axis_index("c")
    s = lax.axis_index("s")
    f0 = s * fpt
    ebase = c * (E // 2)
    nchunk = (E // 2) // CH
    out_off = c * (fdim * N)

    pltpu.async_copy(packedE.at[pl.ds(ebase, CH)], pbufs[0], sems[0])

    @pl.when(f0 < 32)
    def _():
        pltpu.sync_copy(hbits0, wtab)

    @pl.when(f0 >= 32)
    def _():
        pltpu.sync_copy(hbits1, wtab)

    fb0 = lax.rem(f0, 32)

    def zero_body(i, _):
        aggcol[pl.ds(i * 16, 16)] = jnp.zeros((16,), jnp.float32)
        return 0

    lax.fori_loop(0, fpt * N // 16, zero_body, 0)

    ones = jnp.ones((16,), jnp.float32)

    def chunk_pair(ci2, _):
        for b in range(2):
            ci = ci2 * 2 + b
            pltpu.make_async_copy(
                packedE.at[pl.ds(ebase + ci * CH, CH)], pbufs[b], sems[b]
            ).wait()

            @pl.when(ci + 1 < nchunk)
            def _():
                pltpu.async_copy(
                    packedE.at[pl.ds(ebase + (ci + 1) * CH, CH)],
                    pbufs[1 - b], sems[1 - b])

            def grp(g, _):
                masks = []
                dsts = []
                for u in range(2):
                    p16 = pbufs[b][pl.ds(g * 32 + u * 16, 16)]
                    s16 = lax.bitwise_and(p16, (1 << PACK_SHIFT) - 1)
                    d16 = lax.shift_right_logical(p16, PACK_SHIFT)
                    dsts.append(d16)
                    w16 = plsc.load_gather(wtab, [s16])
                    wsh = lax.shift_right_logical(
                        w16, lax.broadcast_in_dim(fb0, (16,), ()))
                    for f in range(fpt):
                        bit = lax.bitwise_and(
                            lax.shift_right_logical(wsh, f), 1)
                        masks.append(bit == 1)
                for u in range(2):
                    for f in range(fpt):
                        plsc.addupdate_scatter(
                            aggcol, [dsts[u] + (f * N)], ones,
                            mask=masks[u * fpt + f])
                return 0

            lax.fori_loop(0, CH // 32, grp, 0)
        return 0

    lax.fori_loop(0, nchunk // 2, chunk_pair, 0)
    pltpu.sync_copy(aggcol, aggT.at[pl.ds(out_off + f0 * N, fpt * N)])


def _sc_aggregate_bits(hbits0, hbits1, packed):
    fpt = H // NSC
    mesh = plsc.VectorSubcoreMesh(core_axis_name="c", subcore_axis_name="s",
                                  num_cores=2, num_subcores=16)
    run = pl.kernel(
        _sc_agg_bits_body,
        out_type=jax.ShapeDtypeStruct((2 * H * N,), jnp.float32),
        mesh=mesh,
        scratch_types=[
            pltpu.VMEM((N,), jnp.int32),
            pltpu.VMEM((fpt * N,), jnp.float32),
            pltpu.VMEM((CH,), jnp.int32),
            pltpu.VMEM((CH,), jnp.int32),
            pltpu.SemaphoreType.DMA,
            pltpu.SemaphoreType.DMA,
        ],
        compiler_params=pltpu.CompilerParams(needs_layout_passes=False),
    )
    return run(packed, hbits0, hbits1).reshape(2, H, N)


def _sc_xload_body(fdim, split_cores, xT, xcol):
    c = lax.axis_index("c")
    s = lax.axis_index("s")
    if split_cores:
        f0 = s * (fdim // NSC)
        fpt = fdim // NSC
    else:
        f0 = (s * 2 + c) * (fdim // NW)
        fpt = fdim // NW
    pltpu.sync_copy(xT.at[pl.ds(f0 * N, fpt * N)], xcol)


def _sc_aggregate(xT, packed, split_cores):
    fdim = xT.shape[0]
    fpt = fdim // (NSC if split_cores else NW)
    ncopies = 2 if split_cores else 1
    mesh = plsc.VectorSubcoreMesh(core_axis_name="c", subcore_axis_name="s",
                                  num_cores=2, num_subcores=16)

    def body(xT_hbm, packedE, aggT, xcol, aggcol, pbuf0, pbuf1, sem0, sem1):
        _sc_xload_body(fdim, split_cores, xT_hbm, xcol)
        _sc_agg_body(fdim, split_cores, packedE, aggT, xcol, aggcol,
                     (pbuf0, pbuf1), (sem0, sem1))

    run = pl.kernel(
        body,
        out_type=jax.ShapeDtypeStruct((ncopies * fdim * N,), jnp.float32),
        mesh=mesh,
        scratch_types=[
            pltpu.VMEM((fpt * N,), jnp.float32),
            pltpu.VMEM((fpt * N,), jnp.float32),
            pltpu.VMEM((CH,), jnp.int32),
            pltpu.VMEM((CH,), jnp.int32),
            pltpu.SemaphoreType.DMA,
            pltpu.SemaphoreType.DMA,
        ],
        compiler_params=pltpu.CompilerParams(needs_layout_passes=False),
    )
    out = run(xT.reshape(fdim * N), packed)
    if split_cores:
        return out.reshape(2, fdim, N)
    return out.reshape(1, fdim, N)


def _pack_body(s_ref, d_ref, p_ref):
    p_ref[...] = jnp.bitwise_or(s_ref[...],
                                jnp.left_shift(d_ref[...], PACK_SHIFT))


def _pack_edges(src, dst):
    s2 = src.reshape(E // 128, 128)
    d2 = dst.reshape(E // 128, 128)
    p = pl.pallas_call(
        _pack_body,
        in_specs=[
            pl.BlockSpec((E // 128, 128), lambda: (0, 0)),
            pl.BlockSpec((E // 128, 128), lambda: (0, 0)),
        ],
        out_specs=pl.BlockSpec((E // 128, 128), lambda: (0, 0)),
        out_shape=jax.ShapeDtypeStruct((E // 128, 128), jnp.int32),
    )(s2, d2)
    return p.reshape(E)


def _transpose_body(x_ref, xT_ref):
    xT_ref[...] = x_ref[...].T


def _transpose(x):
    n, f = x.shape
    grid = (n + ROW_BLK - 1) // ROW_BLK
    return pl.pallas_call(
        _transpose_body,
        grid=(grid,),
        in_specs=[pl.BlockSpec((ROW_BLK, f), lambda i: (i, 0))],
        out_specs=pl.BlockSpec((f, ROW_BLK), lambda i: (0, i)),
        out_shape=jax.ShapeDtypeStruct((f, n), jnp.float32),
    )(x)


def _layer_body(nagg, x_ref, agg_ref, w1_ref, b1_ref, w2_ref, b2_ref,
                h_ref, hb0_ref, hb1_ref):
    agg = agg_ref[...]
    aggT = agg[0]
    for a in range(1, nagg):
        aggT = aggT + agg[a]
    u = (1.0 + EPS) * x_ref[...] + aggT.T
    t1 = jnp.dot(u, w1_ref[...], preferred_element_type=jnp.float32) + b1_ref[...]
    z = jnp.maximum(t1, 0.0)
    t2 = jnp.dot(z, w2_ref[...], preferred_element_type=jnp.float32) + b2_ref[...]
    s = jax.nn.sigmoid(t2)
    h = (s > 0.5).astype(jnp.float32)
    h_ref[...] = h
    hi = (s > 0.5).astype(jnp.int32)
    sh = lax.rem(jax.lax.broadcasted_iota(jnp.int32, hi.shape, 1), 32)
    bits = lax.shift_left(hi, sh)
    hb0_ref[...] = jnp.sum(bits[:, :32], axis=1)
    hb1_ref[...] = jnp.sum(bits[:, 32:], axis=1)


def _layer_mlp(x, aggTs, w1, b1, w2, b2):
    f = x.shape[1]
    nagg = aggTs.shape[0]
    grid = (N + ROW_BLK - 1) // ROW_BLK
    return pl.pallas_call(
        functools.partial(_layer_body, nagg),
        grid=(grid,),
        in_specs=[
            pl.BlockSpec((ROW_BLK, f), lambda i: (i, 0)),
            pl.BlockSpec((nagg, f, ROW_BLK), lambda i: (0, 0, i)),
            pl.BlockSpec((f, H), lambda i: (0, 0)),
            pl.BlockSpec((H,), lambda i: (0,)),
            pl.BlockSpec((H, H), lambda i: (0, 0)),
            pl.BlockSpec((H,), lambda i: (0,)),
        ],
        out_specs=[
            pl.BlockSpec((ROW_BLK, H), lambda i: (i, 0)),
            pl.BlockSpec((ROW_BLK,), lambda i: (i,)),
            pl.BlockSpec((ROW_BLK,), lambda i: (i,)),
        ],
        out_shape=[
            jax.ShapeDtypeStruct((N, H), jnp.float32),
            jax.ShapeDtypeStruct((N,), jnp.int32),
            jax.ShapeDtypeStruct((N,), jnp.int32),
        ],
    )(x, aggTs, w1, b1, w2, b2)


def _head_body(h0_ref, h1_ref, h2_ref, fc1w_ref, fc1b_ref, fc2w_ref, fc2b_ref,
               out_ref, sum_ref, max_ref):
    i = pl.program_id(0)
    nblk = pl.num_programs(0)
    xh = jnp.concatenate([h0_ref[...], h1_ref[...], h2_ref[...]], axis=1)
    row = i * ROW_BLK + jax.lax.broadcasted_iota(jnp.int32, xh.shape, 0)
    valid = row < N
    xs = jnp.where(valid, xh, 0.0)
    xm = jnp.where(valid, xh, -jnp.inf)
    psum = jnp.sum(xs, axis=0, keepdims=True)
    pmax = jnp.max(xm, axis=0, keepdims=True)

    @pl.when(i == 0)
    def _():
        sum_ref[...] = psum
        max_ref[...] = pmax

    @pl.when(i > 0)
    def _():
        sum_ref[...] += psum
        max_ref[...] = jnp.maximum(max_ref[...], pmax)

    @pl.when(i == nblk - 1)
    def _():
        tsum = sum_ref[...]
        feat = jnp.concatenate([tsum / N, max_ref[...], tsum], axis=1)
        o1 = jnp.maximum(
            jnp.dot(feat, fc1w_ref[...], preferred_element_type=jnp.float32)
            + fc1b_ref[...], 0.0)
        o2 = (jnp.dot(o1, fc2w_ref[...], preferred_element_type=jnp.float32)
              + fc2b_ref[...])
        out_ref[...] = jax.nn.sigmoid(o2)


def _head(h0, h1, h2, fc1_W, fc1_b, fc2_W, fc2_b):
    grid = (N + ROW_BLK - 1) // ROW_BLK
    out, _, _ = pl.pallas_call(
        _head_body,
        grid=(grid,),
        in_specs=[
            pl.BlockSpec((ROW_BLK, H), lambda i: (i, 0)),
            pl.BlockSpec((ROW_BLK, H), lambda i: (i, 0)),
            pl.BlockSpec((ROW_BLK, H), lambda i: (i, 0)),
            pl.BlockSpec((3 * 3 * H, H), lambda i: (0, 0)),
            pl.BlockSpec((H,), lambda i: (0,)),
            pl.BlockSpec((H, C), lambda i: (0, 0)),
            pl.BlockSpec((C,), lambda i: (0,)),
        ],
        out_specs=[
            pl.BlockSpec((1, C), lambda i: (0, 0)),
            pl.BlockSpec((1, 3 * H), lambda i: (0, 0)),
            pl.BlockSpec((1, 3 * H), lambda i: (0, 0)),
        ],
        out_shape=[
            jax.ShapeDtypeStruct((1, C), jnp.float32),
            jax.ShapeDtypeStruct((1, 3 * H), jnp.float32),
            jax.ShapeDtypeStruct((1, 3 * H), jnp.float32),
        ],
    )(h0, h1, h2, fc1_W, fc1_b, fc2_W, fc2_b)
    return out


def kernel(x, edge_index, W1_0, b1_0, W2_0, b2_0, W1_1, b1_1, W2_1, b2_1,
           W1_2, b1_2, W2_2, b2_2, fc1_W, fc1_b, fc2_W, fc2_b):
    src = edge_index[0]
    dst = edge_index[1]
    packed = _pack_edges(src, dst)
    xT = _transpose(x)
    aggT0 = _sc_aggregate(xT, packed, split_cores=False)
    h0, h0b0, h0b1 = _layer_mlp(x, aggT0, W1_0, b1_0, W2_0, b2_0)
    aggT1 = _sc_aggregate_bits(h0b0, h0b1, packed)
    h1, h1b0, h1b1 = _layer_mlp(h0, aggT1, W1_1, b1_1, W2_1, b2_1)
    aggT2 = _sc_aggregate_bits(h1b0, h1b1, packed)
    h2, _, _ = _layer_mlp(h1, aggT2, W1_2, b1_2, W2_2, b2_2)
    return _head(h0, h1, h2, fc1_W, fc1_b, fc2_W, fc2_b)


# speedup vs baseline: 2.2140x; 1.1991x over previous
import functools

import jax
import jax.numpy as jnp
from jax import lax
from jax.experimental import pallas as pl
from jax.experimental.pallas import tpu as pltpu
from jax.experimental.pallas import tpu_sc as plsc

N = 10000
E = 320000
H = 64
C = 16
EPS = 1.0

ROW_BLK = 1024
CH = 3200
NW = 32
NSC = 16
PACK_SHIFT = 14


def _sc_agg_body(fdim, split_cores, packedE, aggT, xcol, aggcol, pbufs, sems):
    c = lax.axis_index("c")
    s = lax.axis_index("s")
    if split_cores:
        fpt = fdim // NSC
        f0 = s * fpt
        ebase = c * (E // 2)
        nchunk = (E // 2) // CH
        out_off = c * (fdim * N)
    else:
        fpt = fdim // NW
        f0 = (s * 2 + c) * fpt
        ebase = 0
        nchunk = E // CH
        out_off = 0

    pltpu.async_copy(packedE.at[pl.ds(ebase, CH)], pbufs[0], sems[0])

    def zero_body(i, _):
        aggcol[pl.ds(i * 16, 16)] = jnp.zeros((16,), jnp.float32)
        return 0

    lax.fori_loop(0, fpt * N // 16, zero_body, 0)

    def chunk_pair(ci2, _):
        for b in range(2):
            ci = ci2 * 2 + b
            pltpu.make_async_copy(
                packedE.at[pl.ds(ebase + ci * CH, CH)], pbufs[b], sems[b]
            ).wait()

            @pl.when(ci + 1 < nchunk)
            def _():
                pltpu.async_copy(
                    packedE.at[pl.ds(ebase + (ci + 1) * CH, CH)],
                    pbufs[1 - b], sems[1 - b])

            def grp(g, _):
                vals = []
                dsts = []
                for u in range(4):
                    p16 = pbufs[b][pl.ds(g * 64 + u * 16, 16)]
                    s16 = lax.bitwise_and(p16, (1 << PACK_SHIFT) - 1)
                    d16 = lax.shift_right_logical(p16, PACK_SHIFT)
                    dsts.append(d16)
                    for f in range(fpt):
                        vals.append(plsc.load_gather(xcol, [s16 + (f * N)]))
                for u in range(4):
                    for f in range(fpt):
                        plsc.addupdate_scatter(
                            aggcol, [dsts[u] + (f * N)], vals[u * fpt + f])
                return 0

            lax.fori_loop(0, CH // 64, grp, 0)
        return 0

    lax.fori_loop(0, nchunk // 2, chunk_pair, 0)
    pltpu.sync_copy(aggcol, aggT.at[pl.ds(out_off + f0 * N, fpt * N)])


def _sc_agg_bits_body(packedE, hbits0, hbits1, aggT, wtab, aggcol,
                      pbuf0, pbuf1, sem0, sem1):
    fdim = H
    fpt = fdim // NSC
    pbufs = (pbuf0, pbuf1)
    sems = (sem0, sem1)
    c = lax.axis_index("c")
    s = lax.axis_index("s")
    f0 = s * fpt
    ebase = c * (E // 2)
    nchunk = (E // 2) // CH
    out_off = c * (fdim * N)

    pltpu.async_copy(packedE.at[pl.ds(ebase, CH)], pbufs[0], sems[0])

    @pl.when(f0 < 32)
    def _():
        pltpu.sync_copy(hbits0, wtab)

    @pl.when(f0 >= 32)
    def _():
        pltpu.sync_copy(hbits1, wtab)

    fb0 = lax.rem(f0, 32)

    def zero_body(i, _):
        aggcol[pl.ds(i * 16, 16)] = jnp.zeros((16,), jnp.float32)
        return 0

    lax.fori_loop(0, fpt * N // 16, zero_body, 0)

    ones = jnp.ones((16,), jnp.float32)

    def chunk_pair(ci2, _):
        for b in range(2):
            ci = ci2 * 2 + b
            pltpu.make_async_copy(
                packedE.at[pl.ds(ebase + ci * CH, CH)], pbufs[b], sems[b]
            ).wait()

            @pl.when(ci + 1 < nchunk)
            def _():
                pltpu.async_copy(
                    packedE.at[pl.ds(ebase + (ci + 1) * CH, CH)],
                    pbufs[1 - b], sems[1 - b])

            def grp(g, _):
                masks = []
                dsts = []
                for u in range(4):
                    p16 = pbufs[b][pl.ds(g * 64 + u * 16, 16)]
                    s16 = lax.bitwise_and(p16, (1 << PACK_SHIFT) - 1)
                    d16 = lax.shift_right_logical(p16, PACK_SHIFT)
                    dsts.append(d16)
                    w16 = plsc.load_gather(wtab, [s16])
                    wsh = lax.shift_right_logical(
                        w16, lax.broadcast_in_dim(fb0, (16,), ()))
                    for f in range(fpt):
                        bit = lax.bitwise_and(
                            lax.shift_right_logical(wsh, f), 1)
                        masks.append(bit == 1)
                for u in range(4):
                    for f in range(fpt):
                        plsc.addupdate_scatter(
                            aggcol, [dsts[u] + (f * N)], ones,
                            mask=masks[u * fpt + f])
                return 0

            lax.fori_loop(0, CH // 64, grp, 0)
        return 0

    lax.fori_loop(0, nchunk // 2, chunk_pair, 0)
    pltpu.sync_copy(aggcol, aggT.at[pl.ds(out_off + f0 * N, fpt * N)])


def _sc_aggregate_bits(hbits0, hbits1, packed):
    fpt = H // NSC
    mesh = plsc.VectorSubcoreMesh(core_axis_name="c", subcore_axis_name="s",
                                  num_cores=2, num_subcores=16)
    run = pl.kernel(
        _sc_agg_bits_body,
        out_type=jax.ShapeDtypeStruct((2 * H * N,), jnp.float32),
        mesh=mesh,
        scratch_types=[
            pltpu.VMEM((N,), jnp.int32),
            pltpu.VMEM((fpt * N,), jnp.float32),
            pltpu.VMEM((CH,), jnp.int32),
            pltpu.VMEM((CH,), jnp.int32),
            pltpu.SemaphoreType.DMA,
            pltpu.SemaphoreType.DMA,
        ],
        compiler_params=pltpu.CompilerParams(needs_layout_passes=False),
    )
    return run(packed, hbits0, hbits1).reshape(2, H, N)


def _sc_xload_body(fdim, split_cores, xT, xcol):
    c = lax.axis_index("c")
    s = lax.axis_index("s")
    if split_cores:
        f0 = s * (fdim // NSC)
        fpt = fdim // NSC
    else:
        f0 = (s * 2 + c) * (fdim // NW)
        fpt = fdim // NW
    pltpu.sync_copy(xT.at[pl.ds(f0 * N, fpt * N)], xcol)


def _sc_aggregate(xT, packed, split_cores):
    fdim = xT.shape[0]
    fpt = fdim // (NSC if split_cores else NW)
    ncopies = 2 if split_cores else 1
    mesh = plsc.VectorSubcoreMesh(core_axis_name="c", subcore_axis_name="s",
                                  num_cores=2, num_subcores=16)

    def body(xT_hbm, packedE, aggT, xcol, aggcol, pbuf0, pbuf1, sem0, sem1):
        _sc_xload_body(fdim, split_cores, xT_hbm, xcol)
        _sc_agg_body(fdim, split_cores, packedE, aggT, xcol, aggcol,
                     (pbuf0, pbuf1), (sem0, sem1))

    run = pl.kernel(
        body,
        out_type=jax.ShapeDtypeStruct((ncopies * fdim * N,), jnp.float32),
        mesh=mesh,
        scratch_types=[
            pltpu.VMEM((fpt * N,), jnp.float32),
            pltpu.VMEM((fpt * N,), jnp.float32),
            pltpu.VMEM((CH,), jnp.int32),
            pltpu.VMEM((CH,), jnp.int32),
            pltpu.SemaphoreType.DMA,
            pltpu.SemaphoreType.DMA,
        ],
        compiler_params=pltpu.CompilerParams(needs_layout_passes=False),
    )
    out = run(xT.reshape(fdim * N), packed)
    if split_cores:
        return out.reshape(2, fdim, N)
    return out.reshape(1, fdim, N)


def _pack_body(s_ref, d_ref, p_ref):
    p_ref[...] = jnp.bitwise_or(s_ref[...],
                                jnp.left_shift(d_ref[...], PACK_SHIFT))


def _pack_edges(src, dst):
    s2 = src.reshape(E // 128, 128)
    d2 = dst.reshape(E // 128, 128)
    p = pl.pallas_call(
        _pack_body,
        in_specs=[
            pl.BlockSpec((E // 128, 128), lambda: (0, 0)),
            pl.BlockSpec((E // 128, 128), lambda: (0, 0)),
        ],
        out_specs=pl.BlockSpec((E // 128, 128), lambda: (0, 0)),
        out_shape=jax.ShapeDtypeStruct((E // 128, 128), jnp.int32),
    )(s2, d2)
    return p.reshape(E)


def _transpose_body(x_ref, xT_ref):
    xT_ref[...] = x_ref[...].T


def _transpose(x):
    n, f = x.shape
    grid = (n + ROW_BLK - 1) // ROW_BLK
    return pl.pallas_call(
        _transpose_body,
        grid=(grid,),
        in_specs=[pl.BlockSpec((ROW_BLK, f), lambda i: (i, 0))],
        out_specs=pl.BlockSpec((f, ROW_BLK), lambda i: (0, i)),
        out_shape=jax.ShapeDtypeStruct((f, n), jnp.float32),
    )(x)


def _layer_body(nagg, x_ref, agg_ref, w1_ref, b1_ref, w2_ref, b2_ref,
                h_ref, hb0_ref, hb1_ref):
    agg = agg_ref[...]
    aggT = agg[0]
    for a in range(1, nagg):
        aggT = aggT + agg[a]
    u = (1.0 + EPS) * x_ref[...] + aggT.T
    t1 = jnp.dot(u, w1_ref[...], preferred_element_type=jnp.float32) + b1_ref[...]
    z = jnp.maximum(t1, 0.0)
    t2 = jnp.dot(z, w2_ref[...], preferred_element_type=jnp.float32) + b2_ref[...]
    s = jax.nn.sigmoid(t2)
    h = (s > 0.5).astype(jnp.float32)
    h_ref[...] = h
    hi = (s > 0.5).astype(jnp.int32)
    sh = lax.rem(jax.lax.broadcasted_iota(jnp.int32, hi.shape, 1), 32)
    bits = lax.shift_left(hi, sh)
    hb0_ref[...] = jnp.sum(bits[:, :32], axis=1)
    hb1_ref[...] = jnp.sum(bits[:, 32:], axis=1)


def _layer_mlp(x, aggTs, w1, b1, w2, b2):
    f = x.shape[1]
    nagg = aggTs.shape[0]
    grid = (N + ROW_BLK - 1) // ROW_BLK
    return pl.pallas_call(
        functools.partial(_layer_body, nagg),
        grid=(grid,),
        in_specs=[
            pl.BlockSpec((ROW_BLK, f), lambda i: (i, 0)),
            pl.BlockSpec((nagg, f, ROW_BLK), lambda i: (0, 0, i)),
            pl.BlockSpec((f, H), lambda i: (0, 0)),
            pl.BlockSpec((H,), lambda i: (0,)),
            pl.BlockSpec((H, H), lambda i: (0, 0)),
            pl.BlockSpec((H,), lambda i: (0,)),
        ],
        out_specs=[
            pl.BlockSpec((ROW_BLK, H), lambda i: (i, 0)),
            pl.BlockSpec((ROW_BLK,), lambda i: (i,)),
            pl.BlockSpec((ROW_BLK,), lambda i: (i,)),
        ],
        out_shape=[
            jax.ShapeDtypeStruct((N, H), jnp.float32),
            jax.ShapeDtypeStruct((N,), jnp.int32),
            jax.ShapeDtypeStruct((N,), jnp.int32),
        ],
    )(x, aggTs, w1, b1, w2, b2)


def _head_body(h0_ref, h1_ref, h2_ref, fc1w_ref, fc1b_ref, fc2w_ref, fc2b_ref,
               out_ref, sum_ref, max_ref):
    i = pl.program_id(0)
    nblk = pl.num_programs(0)
    xh = jnp.concatenate([h0_ref[...], h1_ref[...], h2_ref[...]], axis=1)
    row = i * ROW_BLK + jax.lax.broadcasted_iota(jnp.int32, xh.shape, 0)
    valid = row < N
    xs = jnp.where(valid, xh, 0.0)
    xm = jnp.where(valid, xh, -jnp.inf)
    psum = jnp.sum(xs, axis=0, keepdims=True)
    pmax = jnp.max(xm, axis=0, keepdims=True)

    @pl.when(i == 0)
    def _():
        sum_ref[...] = psum
        max_ref[...] = pmax

    @pl.when(i > 0)
    def _():
        sum_ref[...] += psum
        max_ref[...] = jnp.maximum(max_ref[...], pmax)

    @pl.when(i == nblk - 1)
    def _():
        tsum = sum_ref[...]
        feat = jnp.concatenate([tsum / N, max_ref[...], tsum], axis=1)
        o1 = jnp.maximum(
            jnp.dot(feat, fc1w_ref[...], preferred_element_type=jnp.float32)
            + fc1b_ref[...], 0.0)
        o2 = (jnp.dot(o1, fc2w_ref[...], preferred_element_type=jnp.float32)
              + fc2b_ref[...])
        out_ref[...] = jax.nn.sigmoid(o2)


def _head(h0, h1, h2, fc1_W, fc1_b, fc2_W, fc2_b):
    grid = (N + ROW_BLK - 1) // ROW_BLK
    out, _, _ = pl.pallas_call(
        _head_body,
        grid=(grid,),
        in_specs=[
            pl.BlockSpec((ROW_BLK, H), lambda i: (i, 0)),
            pl.BlockSpec((ROW_BLK, H), lambda i: (i, 0)),
            pl.BlockSpec((ROW_BLK, H), lambda i: (i, 0)),
            pl.BlockSpec((3 * 3 * H, H), lambda i: (0, 0)),
            pl.BlockSpec((H,), lambda i: (0,)),
            pl.BlockSpec((H, C), lambda i: (0, 0)),
            pl.BlockSpec((C,), lambda i: (0,)),
        ],
        out_specs=[
            pl.BlockSpec((1, C), lambda i: (0, 0)),
            pl.BlockSpec((1, 3 * H), lambda i: (0, 0)),
            pl.BlockSpec((1, 3 * H), lambda i: (0, 0)),
        ],
        out_shape=[
            jax.ShapeDtypeStruct((1, C), jnp.float32),
            jax.ShapeDtypeStruct((1, 3 * H), jnp.float32),
            jax.ShapeDtypeStruct((1, 3 * H), jnp.float32),
        ],
    )(h0, h1, h2, fc1_W, fc1_b, fc2_W, fc2_b)
    return out


def kernel(x, edge_index, W1_0, b1_0, W2_0, b2_0, W1_1, b1_1, W2_1, b2_1,
           W1_2, b1_2, W2_2, b2_2, fc1_W, fc1_b, fc2_W, fc2_b):
    src = edge_index[0]
    dst = edge_index[1]
    packed = _pack_edges(src, dst)
    xT = _transpose(x)
    aggT0 = _sc_aggregate(xT, packed, split_cores=False)
    h0, h0b0, h0b1 = _layer_mlp(x, aggT0, W1_0, b1_0, W2_0, b2_0)
    aggT1 = _sc_aggregate_bits(h0b0, h0b1, packed)
    h1, h1b0, h1b1 = _layer_mlp(h0, aggT1, W1_1, b1_1, W2_1, b2_1)
    aggT2 = _sc_aggregate_bits(h1b0, h1b1, packed)
    h2, _, _ = _layer_mlp(h1, aggT2, W1_2, b1_2, W2_2, b2_2)
    return _head(h0, h1, h2, fc1_W, fc1_b, fc2_W, fc2_b)


# speedup vs baseline: 2.3384x; 1.0562x over previous
import functools

import jax
import jax.numpy as jnp
from jax import lax
from jax.experimental import pallas as pl
from jax.experimental.pallas import tpu as pltpu
from jax.experimental.pallas import tpu_sc as plsc

N = 10000
E = 320000
H = 64
C = 16
EPS = 1.0

ROW_BLK = 1024
CH = 3200
NW = 32
NSC = 16
PACK_SHIFT = 14


def _sc_agg_body(fdim, split_cores, packedE, aggT, xcol, aggcol, pbufs, sems):
    c = lax.axis_index("c")
    s = lax.axis_index("s")
    if split_cores:
        fpt = fdim // NSC
        f0 = s * fpt
        ebase = c * (E // 2)
        nchunk = (E // 2) // CH
        out_off = c * (fdim * N)
    else:
        fpt = fdim // NW
        f0 = (s * 2 + c) * fpt
        ebase = 0
        nchunk = E // CH
        out_off = 0

    pltpu.async_copy(packedE.at[pl.ds(ebase, CH)], pbufs[0], sems[0])

    def zero_body(i, _):
        aggcol[pl.ds(i * 16, 16)] = jnp.zeros((16,), jnp.float32)
        return 0

    lax.fori_loop(0, fpt * N // 16, zero_body, 0)

    def chunk_pair(ci2, _):
        for b in range(2):
            ci = ci2 * 2 + b
            pltpu.make_async_copy(
                packedE.at[pl.ds(ebase + ci * CH, CH)], pbufs[b], sems[b]
            ).wait()

            @pl.when(ci + 1 < nchunk)
            def _():
                pltpu.async_copy(
                    packedE.at[pl.ds(ebase + (ci + 1) * CH, CH)],
                    pbufs[1 - b], sems[1 - b])

            def grp(g, _):
                vals = []
                dsts = []
                for u in range(8):
                    p16 = pbufs[b][pl.ds(g * 128 + u * 16, 16)]
                    s16 = lax.bitwise_and(p16, (1 << PACK_SHIFT) - 1)
                    d16 = lax.shift_right_logical(p16, PACK_SHIFT)
                    dsts.append(d16)
                    for f in range(fpt):
                        vals.append(plsc.load_gather(xcol, [s16 + (f * N)]))
                for u in range(8):
                    for f in range(fpt):
                        plsc.addupdate_scatter(
                            aggcol, [dsts[u] + (f * N)], vals[u * fpt + f])
                return 0

            lax.fori_loop(0, CH // 128, grp, 0)
        return 0

    lax.fori_loop(0, nchunk // 2, chunk_pair, 0)
    pltpu.sync_copy(aggcol, aggT.at[pl.ds(out_off + f0 * N, fpt * N)])


def _sc_agg_bits_body(packedE, hbits0, hbits1, aggT, wtab, aggcol,
                      pbuf0, pbuf1, sem0, sem1):
    fdim = H
    fpt = fdim // NSC
    pbufs = (pbuf0, pbuf1)
    sems = (sem0, sem1)
    c = lax.axis_index("c")
    s = lax.axis_index("s")
    f0 = s * fpt
    ebase = c * (E // 2)
    nchunk = (E // 2) // CH
    out_off = c * (fdim * N)

    pltpu.async_copy(packedE.at[pl.ds(ebase, CH)], pbufs[0], sems[0])

    @pl.when(f0 < 32)
    def _():
        pltpu.sync_copy(hbits0, wtab)

    @pl.when(f0 >= 32)
    def _():
        pltpu.sync_copy(hbits1, wtab)

    fb0 = lax.rem(f0, 32)

    def zero_body(i, _):
        aggcol[pl.ds(i * 16, 16)] = jnp.zeros((16,), jnp.float32)
        return 0

    lax.fori_loop(0, fpt * N // 16, zero_body, 0)

    ones = jnp.ones((16,), jnp.float32)

    def chunk_pair(ci2, _):
        for b in range(2):
            ci = ci2 * 2 + b
            pltpu.make_async_copy(
                packedE.at[pl.ds(ebase + ci * CH, CH)], pbufs[b], sems[b]
            ).wait()

            @pl.when(ci + 1 < nchunk)
            def _():
                pltpu.async_copy(
                    packedE.at[pl.ds(ebase + (ci + 1) * CH, CH)],
                    pbufs[1 - b], sems[1 - b])

            def grp(g, _):
                masks = []
                dsts = []
                for u in range(4):
                    p16 = pbufs[b][pl.ds(g * 64 + u * 16, 16)]
                    s16 = lax.bitwise_and(p16, (1 << PACK_SHIFT) - 1)
                    d16 = lax.shift_right_logical(p16, PACK_SHIFT)
                    dsts.append(d16)
                    w16 = plsc.load_gather(wtab, [s16])
                    wsh = lax.shift_right_logical(
                        w16, lax.broadcast_in_dim(fb0, (16,), ()))
                    for f in range(fpt):
                        bit = lax.bitwise_and(
                            lax.shift_right_logical(wsh, f), 1)
                        masks.append(bit == 1)
                for u in range(4):
                    for f in range(fpt):
                        plsc.addupdate_scatter(
                            aggcol, [dsts[u] + (f * N)], ones,
                            mask=masks[u * fpt + f])
                return 0

            lax.fori_loop(0, CH // 64, grp, 0)
        return 0

    lax.fori_loop(0, nchunk // 2, chunk_pair, 0)
    pltpu.sync_copy(aggcol, aggT.at[pl.ds(out_off + f0 * N, fpt * N)])


def _sc_aggregate_bits(hbits0, hbits1, packed):
    fpt = H // NSC
    mesh = plsc.VectorSubcoreMesh(core_axis_name="c", subcore_axis_name="s",
                                  num_cores=2, num_subcores=16)
    run = pl.kernel(
        _sc_agg_bits_body,
        out_type=jax.ShapeDtypeStruct((2 * H * N,), jnp.float32),
        mesh=mesh,
        scratch_types=[
            pltpu.VMEM((N,), jnp.int32),
            pltpu.VMEM((fpt * N,), jnp.float32),
            pltpu.VMEM((CH,), jnp.int32),
            pltpu.VMEM((CH,), jnp.int32),
            pltpu.SemaphoreType.DMA,
            pltpu.SemaphoreType.DMA,
        ],
        compiler_params=pltpu.CompilerParams(needs_layout_passes=False),
    )
    return run(packed, hbits0, hbits1).reshape(2, H, N)


def _sc_xload_body(fdim, split_cores, xT, xcol):
    c = lax.axis_index("c")
    s = lax.axis_index("s")
    if split_cores:
        f0 = s * (fdim // NSC)
        fpt = fdim // NSC
    else:
        f0 = (s * 2 + c) * (fdim // NW)
        fpt = fdim // NW
    pltpu.sync_copy(xT.at[pl.ds(f0 * N, fpt * N)], xcol)


def _sc_aggregate(xT, packed, split_cores):
    fdim = xT.shape[0]
    fpt = fdim // (NSC if split_cores else NW)
    ncopies = 2 if split_cores else 1
    mesh = plsc.VectorSubcoreMesh(core_axis_name="c", subcore_axis_name="s",
                                  num_cores=2, num_subcores=16)

    def body(xT_hbm, packedE, aggT, xcol, aggcol, pbuf0, pbuf1, sem0, sem1):
        _sc_xload_body(fdim, split_cores, xT_hbm, xcol)
        _sc_agg_body(fdim, split_cores, packedE, aggT, xcol, aggcol,
                     (pbuf0, pbuf1), (sem0, sem1))

    run = pl.kernel(
        body,
        out_type=jax.ShapeDtypeStruct((ncopies * fdim * N,), jnp.float32),
        mesh=mesh,
        scratch_types=[
            pltpu.VMEM((fpt * N,), jnp.float32),
            pltpu.VMEM((fpt * N,), jnp.float32),
            pltpu.VMEM((CH,), jnp.int32),
            pltpu.VMEM((CH,), jnp.int32),
            pltpu.SemaphoreType.DMA,
            pltpu.SemaphoreType.DMA,
        ],
        compiler_params=pltpu.CompilerParams(needs_layout_passes=False),
    )
    out = run(xT.reshape(fdim * N), packed)
    if split_cores:
        return out.reshape(2, fdim, N)
    return out.reshape(1, fdim, N)


def _prep_body(x_ref, s_ref, d_ref, xT_ref, p_ref):
    xT_ref[...] = x_ref[...].T
    p_ref[...] = jnp.bitwise_or(s_ref[...],
                                jnp.left_shift(d_ref[...], PACK_SHIFT))


def _prep(x, src, dst):
    n, f = x.shape
    grid = (n + ROW_BLK - 1) // ROW_BLK
    s2 = src.reshape(E // 128, 128)
    d2 = dst.reshape(E // 128, 128)
    xT, p = pl.pallas_call(
        _prep_body,
        grid=(grid,),
        in_specs=[
            pl.BlockSpec((ROW_BLK, f), lambda i: (i, 0)),
            pl.BlockSpec((E // 128, 128), lambda i: (0, 0)),
            pl.BlockSpec((E // 128, 128), lambda i: (0, 0)),
        ],
        out_specs=[
            pl.BlockSpec((f, ROW_BLK), lambda i: (0, i)),
            pl.BlockSpec((E // 128, 128), lambda i: (0, 0)),
        ],
        out_shape=[
            jax.ShapeDtypeStruct((f, n), jnp.float32),
            jax.ShapeDtypeStruct((E // 128, 128), jnp.int32),
        ],
    )(x, s2, d2)
    return xT, p.reshape(E)


def _layer_body(nagg, x_ref, agg_ref, w1_ref, b1_ref, w2_ref, b2_ref,
                h_ref, hb0_ref, hb1_ref):
    agg = agg_ref[...]
    aggT = agg[0]
    for a in range(1, nagg):
        aggT = aggT + agg[a]
    u = (1.0 + EPS) * x_ref[...] + aggT.T
    t1 = jnp.dot(u, w1_ref[...], preferred_element_type=jnp.float32) + b1_ref[...]
    z = jnp.maximum(t1, 0.0)
    t2 = jnp.dot(z, w2_ref[...], preferred_element_type=jnp.float32) + b2_ref[...]
    s = jax.nn.sigmoid(t2)
    h = (s > 0.5).astype(jnp.float32)
    h_ref[...] = h
    hi = (s > 0.5).astype(jnp.int32)
    sh = lax.rem(jax.lax.broadcasted_iota(jnp.int32, hi.shape, 1), 32)
    bits = lax.shift_left(hi, sh)
    hb0_ref[...] = jnp.sum(bits[:, :32], axis=1)
    hb1_ref[...] = jnp.sum(bits[:, 32:], axis=1)


def _layer_mlp(x, aggTs, w1, b1, w2, b2):
    f = x.shape[1]
    nagg = aggTs.shape[0]
    grid = (N + ROW_BLK - 1) // ROW_BLK
    return pl.pallas_call(
        functools.partial(_layer_body, nagg),
        grid=(grid,),
        in_specs=[
            pl.BlockSpec((ROW_BLK, f), lambda i: (i, 0)),
            pl.BlockSpec((nagg, f, ROW_BLK), lambda i: (0, 0, i)),
            pl.BlockSpec((f, H), lambda i: (0, 0)),
            pl.BlockSpec((H,), lambda i: (0,)),
            pl.BlockSpec((H, H), lambda i: (0, 0)),
            pl.BlockSpec((H,), lambda i: (0,)),
        ],
        out_specs=[
            pl.BlockSpec((ROW_BLK, H), lambda i: (i, 0)),
            pl.BlockSpec((ROW_BLK,), lambda i: (i,)),
            pl.BlockSpec((ROW_BLK,), lambda i: (i,)),
        ],
        out_shape=[
            jax.ShapeDtypeStruct((N, H), jnp.float32),
            jax.ShapeDtypeStruct((N,), jnp.int32),
            jax.ShapeDtypeStruct((N,), jnp.int32),
        ],
    )(x, aggTs, w1, b1, w2, b2)


def _layer2_head_body(x_ref, agg_ref, w1_ref, b1_ref, w2_ref, b2_ref,
                      h0_ref, fc1w_ref, fc1b_ref, fc2w_ref, fc2b_ref,
                      out_ref, sum_ref, max_ref):
    i = pl.program_id(0)
    nblk = pl.num_programs(0)
    agg = agg_ref[...]
    aggT = agg[0] + agg[1]
    u = (1.0 + EPS) * x_ref[...] + aggT.T
    t1 = jnp.dot(u, w1_ref[...], preferred_element_type=jnp.float32) + b1_ref[...]
    z = jnp.maximum(t1, 0.0)
    t2 = jnp.dot(z, w2_ref[...], preferred_element_type=jnp.float32) + b2_ref[...]
    h2 = (jax.nn.sigmoid(t2) > 0.5).astype(jnp.float32)

    xh = jnp.concatenate([h0_ref[...], x_ref[...], h2], axis=1)
    row = i * ROW_BLK + jax.lax.broadcasted_iota(jnp.int32, xh.shape, 0)
    valid = row < N
    xs = jnp.where(valid, xh, 0.0)
    xm = jnp.where(valid, xh, -jnp.inf)
    psum = jnp.sum(xs, axis=0, keepdims=True)
    pmax = jnp.max(xm, axis=0, keepdims=True)

    @pl.when(i == 0)
    def _():
        sum_ref[...] = psum
        max_ref[...] = pmax

    @pl.when(i > 0)
    def _():
        sum_ref[...] += psum
        max_ref[...] = jnp.maximum(max_ref[...], pmax)

    @pl.when(i == nblk - 1)
    def _():
        tsum = sum_ref[...]
        feat = jnp.concatenate([tsum / N, max_ref[...], tsum], axis=1)
        o1 = jnp.maximum(
            jnp.dot(feat, fc1w_ref[...], preferred_element_type=jnp.float32)
            + fc1b_ref[...], 0.0)
        o2 = (jnp.dot(o1, fc2w_ref[...], preferred_element_type=jnp.float32)
              + fc2b_ref[...])
        out_ref[...] = jax.nn.sigmoid(o2)


def _layer2_head(h1, aggTs, w1, b1, w2, b2, h0, fc1_W, fc1_b, fc2_W, fc2_b):
    grid = (N + ROW_BLK - 1) // ROW_BLK
    out, _, _ = pl.pallas_call(
        _layer2_head_body,
        grid=(grid,),
        in_specs=[
            pl.BlockSpec((ROW_BLK, H), lambda i: (i, 0)),
            pl.BlockSpec((2, H, ROW_BLK), lambda i: (0, 0, i)),
            pl.BlockSpec((H, H), lambda i: (0, 0)),
            pl.BlockSpec((H,), lambda i: (0,)),
            pl.BlockSpec((H, H), lambda i: (0, 0)),
            pl.BlockSpec((H,), lambda i: (0,)),
            pl.BlockSpec((ROW_BLK, H), lambda i: (i, 0)),
            pl.BlockSpec((3 * 3 * H, H), lambda i: (0, 0)),
            pl.BlockSpec((H,), lambda i: (0,)),
            pl.BlockSpec((H, C), lambda i: (0, 0)),
            pl.BlockSpec((C,), lambda i: (0,)),
        ],
        out_specs=[
            pl.BlockSpec((1, C), lambda i: (0, 0)),
            pl.BlockSpec((1, 3 * H), lambda i: (0, 0)),
            pl.BlockSpec((1, 3 * H), lambda i: (0, 0)),
        ],
        out_shape=[
            jax.ShapeDtypeStruct((1, C), jnp.float32),
            jax.ShapeDtypeStruct((1, 3 * H), jnp.float32),
            jax.ShapeDtypeStruct((1, 3 * H), jnp.float32),
        ],
    )(h1, aggTs, w1, b1, w2, b2, h0, fc1_W, fc1_b, fc2_W, fc2_b)
    return out


def kernel(x, edge_index, W1_0, b1_0, W2_0, b2_0, W1_1, b1_1, W2_1, b2_1,
           W1_2, b1_2, W2_2, b2_2, fc1_W, fc1_b, fc2_W, fc2_b):
    src = edge_index[0]
    dst = edge_index[1]
    xT, packed = _prep(x, src, dst)
    aggT0 = _sc_aggregate(xT, packed, split_cores=False)
    h0, h0b0, h0b1 = _layer_mlp(x, aggT0, W1_0, b1_0, W2_0, b2_0)
    aggT1 = _sc_aggregate_bits(h0b0, h0b1, packed)
    h1, h1b0, h1b1 = _layer_mlp(h0, aggT1, W1_1, b1_1, W2_1, b2_1)
    aggT2 = _sc_aggregate_bits(h1b0, h1b1, packed)
    return _layer2_head(h1, aggT2, W1_2, b1_2, W2_2, b2_2, h0,
                        fc1_W, fc1_b, fc2_W, fc2_b)


# speedup vs baseline: 2.8376x; 1.2135x over previous
import functools

import jax
import jax.numpy as jnp
from jax import lax
from jax.experimental import pallas as pl
from jax.experimental.pallas import tpu as pltpu
from jax.experimental.pallas import tpu_sc as plsc

N = 10000
E = 320000
H = 64
C = 16
EPS = 1.0

ROW_BLK = 1024
CH = 3200
NW = 32
NSC = 16
PACK_SHIFT = 14


def _sc_agg_body(fdim, split_cores, packedE, aggT, xcol, aggcol, pbufs, sems):
    c = lax.axis_index("c")
    s = lax.axis_index("s")
    if split_cores:
        fpt = fdim // NSC
        f0 = s * fpt
        ebase = c * (E // 2)
        nchunk = (E // 2) // CH
        out_off = c * (fdim * N)
    else:
        fpt = fdim // NW
        f0 = (s * 2 + c) * fpt
        ebase = 0
        nchunk = E // CH
        out_off = 0

    pltpu.async_copy(packedE.at[pl.ds(ebase, CH)], pbufs[0], sems[0])

    def zero_body(i, _):
        aggcol[pl.ds(i * 16, 16)] = jnp.zeros((16,), jnp.float32)
        return 0

    lax.fori_loop(0, fpt * N // 16, zero_body, 0)

    def chunk_pair(ci2, _):
        for b in range(2):
            ci = ci2 * 2 + b
            pltpu.make_async_copy(
                packedE.at[pl.ds(ebase + ci * CH, CH)], pbufs[b], sems[b]
            ).wait()

            @pl.when(ci + 1 < nchunk)
            def _():
                pltpu.async_copy(
                    packedE.at[pl.ds(ebase + (ci + 1) * CH, CH)],
                    pbufs[1 - b], sems[1 - b])

            def grp(g, _):
                vals = []
                dsts = []
                for u in range(8):
                    p16 = pbufs[b][pl.ds(g * 128 + u * 16, 16)]
                    s16 = lax.bitwise_and(p16, (1 << PACK_SHIFT) - 1)
                    d16 = lax.shift_right_logical(p16, PACK_SHIFT)
                    dsts.append(d16)
                    for f in range(fpt):
                        vals.append(plsc.load_gather(xcol, [s16 + (f * N)]))
                for u in range(8):
                    for f in range(fpt):
                        plsc.addupdate_scatter(
                            aggcol, [dsts[u] + (f * N)], vals[u * fpt + f])
                return 0

            lax.fori_loop(0, CH // 128, grp, 0)
        return 0

    lax.fori_loop(0, nchunk // 2, chunk_pair, 0)
    pltpu.sync_copy(aggcol, aggT.at[pl.ds(out_off + f0 * N, fpt * N)])


def _sc_agg_bits_body(packedE, hbits0, hbits1, aggT, wtab, aggcol,
                      pbuf0, pbuf1, sem0, sem1):
    fdim = H
    fpt = fdim // NSC
    pbufs = (pbuf0, pbuf1)
    sems = (sem0, sem1)
    c = lax.axis_index("c")
    s = lax.axis_index("s")
    f0 = s * fpt
    ebase = c * (E // 2)
    nchunk = (E // 2) // CH
    out_off = c * (fdim * N)

    pltpu.async_copy(packedE.at[pl.ds(ebase, CH)], pbufs[0], sems[0])

    @pl.when(f0 < 32)
    def _():
        pltpu.sync_copy(hbits0, wtab)

    @pl.when(f0 >= 32)
    def _():
        pltpu.sync_copy(hbits1, wtab)

    fb0 = lax.rem(f0, 32)

    def zero_body(i, _):
        aggcol[pl.ds(i * 16, 16)] = jnp.zeros((16,), jnp.float32)
        return 0

    lax.fori_loop(0, fpt * N // 16, zero_body, 0)

    ones = jnp.ones((16,), jnp.float32)

    def chunk_pair(ci2, _):
        for b in range(2):
            ci = ci2 * 2 + b
            pltpu.make_async_copy(
                packedE.at[pl.ds(ebase + ci * CH, CH)], pbufs[b], sems[b]
            ).wait()

            @pl.when(ci + 1 < nchunk)
            def _():
                pltpu.async_copy(
                    packedE.at[pl.ds(ebase + (ci + 1) * CH, CH)],
                    pbufs[1 - b], sems[1 - b])

            @functools.partial(plsc.parallel_loop, 0, CH // 64, unroll=2)
            def _grp(g):
                masks = []
                dsts = []
                for u in range(4):
                    p16 = pbufs[b][pl.ds(g * 64 + u * 16, 16)]
                    s16 = lax.bitwise_and(p16, (1 << PACK_SHIFT) - 1)
                    d16 = lax.shift_right_logical(p16, PACK_SHIFT)
                    dsts.append(d16)
                    w16 = plsc.load_gather(wtab, [s16])
                    wsh = lax.shift_right_logical(
                        w16, lax.broadcast_in_dim(fb0, (16,), ()))
                    for f in range(fpt):
                        bit = lax.bitwise_and(
                            lax.shift_right_logical(wsh, f), 1)
                        masks.append(bit == 1)
                for u in range(4):
                    for f in range(fpt):
                        plsc.addupdate_scatter(
                            aggcol, [dsts[u] + (f * N)], ones,
                            mask=masks[u * fpt + f])
        return 0

    lax.fori_loop(0, nchunk // 2, chunk_pair, 0)
    pltpu.sync_copy(aggcol, aggT.at[pl.ds(out_off + f0 * N, fpt * N)])


def _sc_aggregate_bits(hbits0, hbits1, packed):
    fpt = H // NSC
    mesh = plsc.VectorSubcoreMesh(core_axis_name="c", subcore_axis_name="s",
                                  num_cores=2, num_subcores=16)
    run = pl.kernel(
        _sc_agg_bits_body,
        out_type=jax.ShapeDtypeStruct((2 * H * N,), jnp.float32),
        mesh=mesh,
        scratch_types=[
            pltpu.VMEM((N,), jnp.int32),
            pltpu.VMEM((fpt * N,), jnp.float32),
            pltpu.VMEM((CH,), jnp.int32),
            pltpu.VMEM((CH,), jnp.int32),
            pltpu.SemaphoreType.DMA,
            pltpu.SemaphoreType.DMA,
        ],
        compiler_params=pltpu.CompilerParams(needs_layout_passes=False),
    )
    return run(packed, hbits0, hbits1).reshape(2, H, N)


def _sc_xload_body(fdim, split_cores, xT, xcol):
    c = lax.axis_index("c")
    s = lax.axis_index("s")
    if split_cores:
        f0 = s * (fdim // NSC)
        fpt = fdim // NSC
    else:
        f0 = (s * 2 + c) * (fdim // NW)
        fpt = fdim // NW
    pltpu.sync_copy(xT.at[pl.ds(f0 * N, fpt * N)], xcol)


def _sc_aggregate(xT, packed, split_cores):
    fdim = xT.shape[0]
    fpt = fdim // (NSC if split_cores else NW)
    ncopies = 2 if split_cores else 1
    mesh = plsc.VectorSubcoreMesh(core_axis_name="c", subcore_axis_name="s",
                                  num_cores=2, num_subcores=16)

    def body(xT_hbm, packedE, aggT, xcol, aggcol, pbuf0, pbuf1, sem0, sem1):
        _sc_xload_body(fdim, split_cores, xT_hbm, xcol)
        _sc_agg_body(fdim, split_cores, packedE, aggT, xcol, aggcol,
                     (pbuf0, pbuf1), (sem0, sem1))

    run = pl.kernel(
        body,
        out_type=jax.ShapeDtypeStruct((ncopies * fdim * N,), jnp.float32),
        mesh=mesh,
        scratch_types=[
            pltpu.VMEM((fpt * N,), jnp.float32),
            pltpu.VMEM((fpt * N,), jnp.float32),
            pltpu.VMEM((CH,), jnp.int32),
            pltpu.VMEM((CH,), jnp.int32),
            pltpu.SemaphoreType.DMA,
            pltpu.SemaphoreType.DMA,
        ],
        compiler_params=pltpu.CompilerParams(needs_layout_passes=False),
    )
    out = run(xT.reshape(fdim * N), packed)
    if split_cores:
        return out.reshape(2, fdim, N)
    return out.reshape(1, fdim, N)


def _prep_body(x_ref, s_ref, d_ref, xT_ref, p_ref):
    xT_ref[...] = x_ref[...].T
    p_ref[...] = jnp.bitwise_or(s_ref[...],
                                jnp.left_shift(d_ref[...], PACK_SHIFT))


def _prep(x, src, dst):
    n, f = x.shape
    grid = (n + ROW_BLK - 1) // ROW_BLK
    s2 = src.reshape(E // 128, 128)
    d2 = dst.reshape(E // 128, 128)
    xT, p = pl.pallas_call(
        _prep_body,
        grid=(grid,),
        in_specs=[
            pl.BlockSpec((ROW_BLK, f), lambda i: (i, 0)),
            pl.BlockSpec((E // 128, 128), lambda i: (0, 0)),
            pl.BlockSpec((E // 128, 128), lambda i: (0, 0)),
        ],
        out_specs=[
            pl.BlockSpec((f, ROW_BLK), lambda i: (0, i)),
            pl.BlockSpec((E // 128, 128), lambda i: (0, 0)),
        ],
        out_shape=[
            jax.ShapeDtypeStruct((f, n), jnp.float32),
            jax.ShapeDtypeStruct((E // 128, 128), jnp.int32),
        ],
    )(x, s2, d2)
    return xT, p.reshape(E)


def _layer_body(nagg, x_ref, agg_ref, w1_ref, b1_ref, w2_ref, b2_ref,
                h_ref, hb0_ref, hb1_ref):
    agg = agg_ref[...]
    aggT = agg[0]
    for a in range(1, nagg):
        aggT = aggT + agg[a]
    u = (1.0 + EPS) * x_ref[...] + aggT.T
    t1 = jnp.dot(u, w1_ref[...], preferred_element_type=jnp.float32) + b1_ref[...]
    z = jnp.maximum(t1, 0.0)
    t2 = jnp.dot(z, w2_ref[...], preferred_element_type=jnp.float32) + b2_ref[...]
    s = jax.nn.sigmoid(t2)
    h = (s > 0.5).astype(jnp.float32)
    h_ref[...] = h
    hi = (s > 0.5).astype(jnp.int32)
    sh = lax.rem(jax.lax.broadcasted_iota(jnp.int32, hi.shape, 1), 32)
    bits = lax.shift_left(hi, sh)
    hb0_ref[...] = jnp.sum(bits[:, :32], axis=1)
    hb1_ref[...] = jnp.sum(bits[:, 32:], axis=1)


def _layer_mlp(x, aggTs, w1, b1, w2, b2):
    f = x.shape[1]
    nagg = aggTs.shape[0]
    grid = (N + ROW_BLK - 1) // ROW_BLK
    return pl.pallas_call(
        functools.partial(_layer_body, nagg),
        grid=(grid,),
        in_specs=[
            pl.BlockSpec((ROW_BLK, f), lambda i: (i, 0)),
            pl.BlockSpec((nagg, f, ROW_BLK), lambda i: (0, 0, i)),
            pl.BlockSpec((f, H), lambda i: (0, 0)),
            pl.BlockSpec((H,), lambda i: (0,)),
            pl.BlockSpec((H, H), lambda i: (0, 0)),
            pl.BlockSpec((H,), lambda i: (0,)),
        ],
        out_specs=[
            pl.BlockSpec((ROW_BLK, H), lambda i: (i, 0)),
            pl.BlockSpec((ROW_BLK,), lambda i: (i,)),
            pl.BlockSpec((ROW_BLK,), lambda i: (i,)),
        ],
        out_shape=[
            jax.ShapeDtypeStruct((N, H), jnp.float32),
            jax.ShapeDtypeStruct((N,), jnp.int32),
            jax.ShapeDtypeStruct((N,), jnp.int32),
        ],
    )(x, aggTs, w1, b1, w2, b2)


def _layer2_head_body(x_ref, agg_ref, w1_ref, b1_ref, w2_ref, b2_ref,
                      h0_ref, fc1w_ref, fc1b_ref, fc2w_ref, fc2b_ref,
                      out_ref, sum_ref, max_ref):
    i = pl.program_id(0)
    nblk = pl.num_programs(0)
    agg = agg_ref[...]
    aggT = agg[0] + agg[1]
    u = (1.0 + EPS) * x_ref[...] + aggT.T
    t1 = jnp.dot(u, w1_ref[...], preferred_element_type=jnp.float32) + b1_ref[...]
    z = jnp.maximum(t1, 0.0)
    t2 = jnp.dot(z, w2_ref[...], preferred_element_type=jnp.float32) + b2_ref[...]
    h2 = (jax.nn.sigmoid(t2) > 0.5).astype(jnp.float32)

    xh = jnp.concatenate([h0_ref[...], x_ref[...], h2], axis=1)
    row = i * ROW_BLK + jax.lax.broadcasted_iota(jnp.int32, xh.shape, 0)
    valid = row < N
    xs = jnp.where(valid, xh, 0.0)
    xm = jnp.where(valid, xh, -jnp.inf)
    psum = jnp.sum(xs, axis=0, keepdims=True)
    pmax = jnp.max(xm, axis=0, keepdims=True)

    @pl.when(i == 0)
    def _():
        sum_ref[...] = psum
        max_ref[...] = pmax

    @pl.when(i > 0)
    def _():
        sum_ref[...] += psum
        max_ref[...] = jnp.maximum(max_ref[...], pmax)

    @pl.when(i == nblk - 1)
    def _():
        tsum = sum_ref[...]
        feat = jnp.concatenate([tsum / N, max_ref[...], tsum], axis=1)
        o1 = jnp.maximum(
            jnp.dot(feat, fc1w_ref[...], preferred_element_type=jnp.float32)
            + fc1b_ref[...], 0.0)
        o2 = (jnp.dot(o1, fc2w_ref[...], preferred_element_type=jnp.float32)
              + fc2b_ref[...])
        out_ref[...] = jax.nn.sigmoid(o2)


def _layer2_head(h1, aggTs, w1, b1, w2, b2, h0, fc1_W, fc1_b, fc2_W, fc2_b):
    grid = (N + ROW_BLK - 1) // ROW_BLK
    out, _, _ = pl.pallas_call(
        _layer2_head_body,
        grid=(grid,),
        in_specs=[
            pl.BlockSpec((ROW_BLK, H), lambda i: (i, 0)),
            pl.BlockSpec((2, H, ROW_BLK), lambda i: (0, 0, i)),
            pl.BlockSpec((H, H), lambda i: (0, 0)),
            pl.BlockSpec((H,), lambda i: (0,)),
            pl.BlockSpec((H, H), lambda i: (0, 0)),
            pl.BlockSpec((H,), lambda i: (0,)),
            pl.BlockSpec((ROW_BLK, H), lambda i: (i, 0)),
            pl.BlockSpec((3 * 3 * H, H), lambda i: (0, 0)),
            pl.BlockSpec((H,), lambda i: (0,)),
            pl.BlockSpec((H, C), lambda i: (0, 0)),
            pl.BlockSpec((C,), lambda i: (0,)),
        ],
        out_specs=[
            pl.BlockSpec((1, C), lambda i: (0, 0)),
            pl.BlockSpec((1, 3 * H), lambda i: (0, 0)),
            pl.BlockSpec((1, 3 * H), lambda i: (0, 0)),
        ],
        out_shape=[
            jax.ShapeDtypeStruct((1, C), jnp.float32),
            jax.ShapeDtypeStruct((1, 3 * H), jnp.float32),
            jax.ShapeDtypeStruct((1, 3 * H), jnp.float32),
        ],
    )(h1, aggTs, w1, b1, w2, b2, h0, fc1_W, fc1_b, fc2_W, fc2_b)
    return out


def kernel(x, edge_index, W1_0, b1_0, W2_0, b2_0, W1_1, b1_1, W2_1, b2_1,
           W1_2, b1_2, W2_2, b2_2, fc1_W, fc1_b, fc2_W, fc2_b):
    src = edge_index[0]
    dst = edge_index[1]
    xT, packed = _prep(x, src, dst)
    aggT0 = _sc_aggregate(xT, packed, split_cores=False)
    h0, h0b0, h0b1 = _layer_mlp(x, aggT0, W1_0, b1_0, W2_0, b2_0)
    aggT1 = _sc_aggregate_bits(h0b0, h0b1, packed)
    h1, h1b0, h1b1 = _layer_mlp(h0, aggT1, W1_1, b1_1, W2_1, b2_1)
    aggT2 = _sc_aggregate_bits(h1b0, h1b1, packed)
    return _layer2_head(h1, aggT2, W1_2, b1_2, W2_2, b2_2, h0,
                        fc1_W, fc1_b, fc2_W, fc2_b)
